# baseline (speedup 1.0000x reference)
import functools

import jax
import jax.numpy as jnp
from jax import lax
from jax.experimental import pallas as pl
from jax.experimental.pallas import tpu as pltpu
import jax.experimental.pallas.tpu_sc as plsc

_NC = 2
_NS = 16
_NW = _NC * _NS
_G = 128


def _mesh():
    return plsc.VectorSubcoreMesh(
        core_axis_name="c", subcore_axis_name="s",
        num_cores=_NC, num_subcores=_NS)


@functools.cache
def _make_deg(npad, ep):
    chunks = ep // (_NW * _G)
    rps = npad // _NS
    zc = rps // _G

    def body(pk_hbm, out_hbm, acc_sh, ones_v, buf_v, dst_all, ssem):
        c = lax.axis_index("c")
        s = lax.axis_index("s")
        wid = c * _NS + s
        pltpu.sync_copy(pk_hbm.at[wid], dst_all)

        def unpk(k, carry):
            for j in range(_G // 16):
                v = dst_all[k, pl.ds(j * 16, 16)]
                dst_all[k, pl.ds(j * 16, 16)] = lax.shift_right_logical(v, 16)
            return carry

        lax.fori_loop(0, chunks, unpk, 0)
        for i in range(_G // 16):
            ones_v[pl.ds(i * 16, 16)] = jnp.ones((16,), jnp.float32)
            buf_v[pl.ds(i * 16, 16)] = jnp.zeros((16,), jnp.float32)
        for j in range(zc):
            pltpu.sync_copy(buf_v, acc_sh.at[pl.ds(s * rps + j * _G, _G)])
        plsc.subcore_barrier()

        def issue(k, carry):
            pltpu.async_copy(ones_v, acc_sh.at[dst_all.at[k]], ssem, add=True)
            return carry

        lax.fori_loop(0, chunks, issue, 0)

        def drain(k, carry):
            pltpu.make_async_copy(ones_v, acc_sh.at[dst_all.at[k]], ssem).wait()
            return carry

        lax.fori_loop(0, chunks, drain, 0)
        plsc.subcore_barrier()
        for j in range(zc):
            r0 = s * rps + j * _G
            pltpu.sync_copy(acc_sh.at[pl.ds(r0, _G)], buf_v)
            pltpu.sync_copy(buf_v, out_hbm.at[pl.ds(c * npad + r0, _G)])

    return pl.kernel(
        body,
        out_type=jax.ShapeDtypeStruct((_NC * npad,), jnp.float32),
        mesh=_mesh(),
        scratch_types=[
            pltpu.VMEM_SHARED((npad,), jnp.float32),
            pltpu.VMEM((_G,), jnp.float32),
            pltpu.VMEM((_G,), jnp.float32),
            pltpu.VMEM((ep // (_NW * _G), _G), jnp.int32),
            pltpu.SemaphoreType.DMA,
        ],
        name=f"gcn_deg_{npad}_{ep}",
    )


@functools.cache
def _make_agg(npad, ep, d, colsplit):
    ce = ep // (_NW * _G)
    chunks = 2 * ce if colsplit else ce
    nbuf = 4 if colsplit else 8
    ngroups = chunks // nbuf
    npairs = ngroups // 2
    rps = npad // _NS
    zc = rps // _G

    def body(g_hbm, pk_hbm, out_hbm,
             acc_sh, bufs, pk_all, sidx, didx, gsem_a, gsem_b, ssem):
        c = lax.axis_index("c")
        s = lax.axis_index("s")
        tab = g_hbm.at[c] if colsplit else g_hbm

        if colsplit:
            pltpu.sync_copy(pk_hbm.at[2 * s], pk_all.at[pl.ds(0, ce)])
            pltpu.sync_copy(pk_hbm.at[2 * s + 1], pk_all.at[pl.ds(ce, ce)])
        else:
            pltpu.sync_copy(pk_hbm.at[c * _NS + s], pk_all)

        def zrow(i, carry):
            for j in range(d // 16):
                bufs[0, 0, i, pl.ds(j * 16, 16)] = jnp.zeros((16,), jnp.float32)
            return carry

        lax.fori_loop(0, _G, zrow, 0)
        for j in range(zc):
            pltpu.sync_copy(bufs.at[0, 0], acc_sh.at[pl.ds(s * rps + j * _G, _G)])
        plsc.subcore_barrier()

        gsems = (gsem_a, gsem_b)
        mask = jnp.int32(0xFFFF)

        def unpack(sl, grp):
            for b in range(nbuf):
                q = grp * nbuf + b
                for j in range(_G // 16):
                    v = pk_all[q, pl.ds(j * 16, 16)]
                    sidx[sl, b, pl.ds(j * 16, 16)] = v & mask
                    didx[sl, b, pl.ds(j * 16, 16)] = lax.shift_right_logical(v, 16)

        def issue_gathers(sl, grp):
            unpack(sl, grp)
            for b in range(nbuf):
                pltpu.async_copy(tab.at[sidx.at[sl, b]], bufs.at[sl, b], gsems[sl])

        def drain_gathers(sl):
            for b in range(nbuf):
                pltpu.make_async_copy(tab.at[sidx.at[sl, b]],
                                      bufs.at[sl, b], gsems[sl]).wait()

        def run_scatters(sl):
            descs = [pltpu.async_copy(bufs.at[sl, b], acc_sh.at[didx.at[sl, b]],
                                      ssem, add=True)
                     for b in range(nbuf)]
            for p_ in descs:
                p_.wait()

        issue_gathers(0, 0)

        def pair(h, carry):
            ga = 2 * h
            gb = 2 * h + 1
            issue_gathers(1, gb)
            drain_gathers(0)
            run_scatters(0)
            issue_gathers(0, (ga + 2) % ngroups)
            drain_gathers(1)
            run_scatters(1)
            return carry

        lax.fori_loop(0, npairs, pair, 0)
        drain_gathers(0)
        plsc.subcore_barrier()
        for j in range(zc):
            r0 = s * rps + j * _G
            pltpu.sync_copy(acc_sh.at[pl.ds(r0, _G)], bufs.at[0, 0])
            pltpu.sync_copy(bufs.at[0, 0], out_hbm.at[c].at[pl.ds(r0, _G)])

    return pl.kernel(
        body,
        out_type=jax.ShapeDtypeStruct((_NC, npad, d), jnp.float32),
        mesh=_mesh(),
        scratch_types=[
            pltpu.VMEM_SHARED((npad, d), jnp.float32),
            pltpu.VMEM((2, nbuf, _G, d), jnp.float32),
            pltpu.VMEM((chunks, _G), jnp.int32),
            pltpu.VMEM((2, nbuf, _G), jnp.int32),
            pltpu.VMEM((2, nbuf, _G), jnp.int32),
            pltpu.SemaphoreType.DMA,
            pltpu.SemaphoreType.DMA,
            pltpu.SemaphoreType.DMA,
        ],
        compiler_params=pltpu.CompilerParams(use_tc_tiling_on_sc=False,
                                             disable_bounds_checks=True),
        name=f"gcn_agg_{npad}_{ep}_{d}_{int(colsplit)}",
    )


_BR = 2048


def _row_spec(br, w):
    return pl.BlockSpec((br, w), lambda i: (i, 0))


def _full_spec(shape):
    return pl.BlockSpec(shape, lambda i: (0,) * len(shape))


def _stage_a(d0, d1, xp):
    npad, f = xp.shape
    fh = f // 2

    def body(d0_r, d1_r, x_r, dis_r, g1_r):
        dis = lax.rsqrt(d0_r[...] + d1_r[...] + 1.0)
        dis_r[...] = dis
        xv = x_r[...] * dis
        g1_r[...] = jnp.stack([xv[:, :fh], xv[:, fh:]])

    return pl.pallas_call(
        body,
        grid=(npad // _BR,),
        in_specs=[_row_spec(_BR, 1), _row_spec(_BR, 1), _row_spec(_BR, f)],
        out_specs=[_row_spec(_BR, 1),
                   pl.BlockSpec((2, _BR, fh), lambda i: (0, i, 0))],
        out_shape=[jax.ShapeDtypeStruct((npad, 1), jnp.float32),
                   jax.ShapeDtypeStruct((2, npad, fh), jnp.float32)],
    )(d0, d1, xp)


def _pair_spec(w):
    return pl.BlockSpec((2, _BR, w), lambda i: (0, i, 0))


def _stage_b(p, g1, dis, W1, b1, W2):
    _, npad, fh = g1.shape
    h1w = W1.shape[1]
    h2w = W2.shape[1]

    def body(p_r, g1_r, dis_r, w1_r, b1_r, w2_r, g2_r):
        ds_ = dis_r[...]
        pv = p_r[...]
        gv = g1_r[...]
        za = ds_ * (pv[0] + gv[0])
        zb = ds_ * (pv[1] + gv[1])
        w1 = w1_r[...]
        h1 = (jnp.dot(za, w1[:fh], preferred_element_type=jnp.float32)
              + jnp.dot(zb, w1[fh:], preferred_element_type=jnp.float32))
        h1 = jnp.maximum(h1 + b1_r[...], 0.0)
        t2 = jnp.dot(h1, w2_r[...], preferred_element_type=jnp.float32)
        g2_r[...] = ds_ * t2

    return pl.pallas_call(
        body,
        grid=(npad // _BR,),
        in_specs=[_pair_spec(fh), _pair_spec(fh),
                  _row_spec(_BR, 1), _full_spec((2 * fh, h1w)),
                  _full_spec((1, h1w)), _full_spec((h1w, h2w))],
        out_specs=_row_spec(_BR, h2w),
        out_shape=jax.ShapeDtypeStruct((npad, h2w), jnp.float32),
    )(p, g1, dis, W1, b1, W2)


def _stage_c(p, g2, dis, b2, W3):
    npad, h2w = g2.shape
    cw = W3.shape[1]

    def body(p_r, g2_r, dis_r, b2_r, w3_r, g3_r):
        pv = p_r[...]
        h2 = dis_r[...] * (pv[0] + pv[1] + g2_r[...]) + b2_r[...]
        h2 = jnp.maximum(h2, 0.0)
        t3 = jnp.dot(h2, w3_r[...], preferred_element_type=jnp.float32)
        g3_r[...] = dis_r[...] * t3

    return pl.pallas_call(
        body,
        grid=(npad // _BR,),
        in_specs=[_pair_spec(h2w), _row_spec(_BR, h2w),
                  _row_spec(_BR, 1), _full_spec((1, h2w)), _full_spec((h2w, cw))],
        out_specs=_row_spec(_BR, cw),
        out_shape=jax.ShapeDtypeStruct((npad, cw), jnp.float32),
    )(p, g2, dis, b2, W3)


def _stage_d(p, g3, dis, b3):
    npad, cw = g3.shape

    def body(p_r, g3_r, dis_r, b3_r, out_r):
        pv = p_r[...]
        out_r[...] = dis_r[...] * (pv[0] + pv[1] + g3_r[...]) + b3_r[...]

    return pl.pallas_call(
        body,
        grid=(npad // _BR,),
        in_specs=[_pair_spec(cw), _row_spec(_BR, cw),
                  _row_spec(_BR, 1), _full_spec((1, cw))],
        out_specs=_row_spec(_BR, cw),
        out_shape=jax.ShapeDtypeStruct((npad, cw), jnp.float32),
    )(p, g3, dis, b3)


def kernel(x, edge_index, W1, b1, W2, b2, W3, b3):
    n, f = x.shape
    e = edge_index.shape[1]

    npad = ((n + 16 + _BR - 1) // _BR) * _BR
    step = _NW * _G
    chunks = (e + step - 1) // step
    chunks = ((chunks + 15) // 16) * 16
    ep = chunks * step

    spare = npad - n
    pad_idx = n + (jnp.arange(ep - e, dtype=jnp.int32) % spare)
    src_flat = jnp.concatenate([edge_index[0], pad_idx])
    dst_flat = jnp.concatenate([edge_index[1], pad_idx])
    pkp = (src_flat | (dst_flat << 16)).reshape(_NW, chunks, _G)
    xp = jnp.pad(x, ((0, npad - n), (0, 0)))
    fh = f // 2

    degs = _make_deg(npad, ep)(pkp)
    d0 = degs[:npad].reshape(npad, 1)
    d1 = degs[npad:].reshape(npad, 1)
    dis, g1 = _stage_a(d0, d1, xp)
    p = _make_agg(npad, ep, fh, True)(g1, pkp)
    g2 = _stage_b(p, g1, dis, W1, b1.reshape(1, -1), W2)

    p = _make_agg(npad, ep, g2.shape[1], False)(g2, pkp)
    g3 = _stage_c(p, g2, dis, b2.reshape(1, -1), W3)

    p = _make_agg(npad, ep, g3.shape[1], False)(g3, pkp)
    out = _stage_d(p, g3, dis, b3.reshape(1, -1))
    return out[:n]


# speedup vs baseline: 1.0601x; 1.0601x over previous
import functools

import jax
import jax.numpy as jnp
from jax import lax
from jax.experimental import pallas as pl
from jax.experimental.pallas import tpu as pltpu
import jax.experimental.pallas.tpu_sc as plsc

_NC = 2
_NS = 16
_NW = _NC * _NS
_G = 128


def _mesh():
    return plsc.VectorSubcoreMesh(
        core_axis_name="c", subcore_axis_name="s",
        num_cores=_NC, num_subcores=_NS)


@functools.cache
def _make_deg(npad, ep):
    chunks = ep // (_NW * _G)
    rps = npad // _NS
    zc = rps // _G

    def body(pk_hbm, out_hbm, acc_sh, ones_v, buf_v, dst_all, ssem):
        c = lax.axis_index("c")
        s = lax.axis_index("s")
        wid = c * _NS + s
        pltpu.sync_copy(pk_hbm.at[wid], dst_all)

        def unpk(k, carry):
            for j in range(_G // 16):
                v = dst_all[k, pl.ds(j * 16, 16)]
                dst_all[k, pl.ds(j * 16, 16)] = lax.shift_right_logical(v, 16)
            return carry

        lax.fori_loop(0, chunks, unpk, 0)
        for i in range(_G // 16):
            ones_v[pl.ds(i * 16, 16)] = jnp.ones((16,), jnp.float32)
            buf_v[pl.ds(i * 16, 16)] = jnp.zeros((16,), jnp.float32)
        for j in range(zc):
            pltpu.sync_copy(buf_v, acc_sh.at[pl.ds(s * rps + j * _G, _G)])
        plsc.subcore_barrier()

        def issue(k, carry):
            pltpu.async_copy(ones_v, acc_sh.at[dst_all.at[k]], ssem, add=True)
            return carry

        lax.fori_loop(0, chunks, issue, 0)

        def drain(k, carry):
            pltpu.make_async_copy(ones_v, acc_sh.at[dst_all.at[k]], ssem).wait()
            return carry

        lax.fori_loop(0, chunks, drain, 0)
        plsc.subcore_barrier()
        for j in range(zc):
            r0 = s * rps + j * _G
            pltpu.sync_copy(acc_sh.at[pl.ds(r0, _G)], buf_v)
            pltpu.sync_copy(buf_v, out_hbm.at[pl.ds(c * npad + r0, _G)])

    return pl.kernel(
        body,
        out_type=jax.ShapeDtypeStruct((_NC * npad,), jnp.float32),
        mesh=_mesh(),
        scratch_types=[
            pltpu.VMEM_SHARED((npad,), jnp.float32),
            pltpu.VMEM((_G,), jnp.float32),
            pltpu.VMEM((_G,), jnp.float32),
            pltpu.VMEM((ep // (_NW * _G), _G), jnp.int32),
            pltpu.SemaphoreType.DMA,
        ],
        name=f"gcn_deg_{npad}_{ep}",
    )


@functools.cache
def _make_agg(npad, ep, d):
    chunks = ep // (_NW * _G)
    nbuf = 1 if d >= 128 else 8
    ngroups = chunks // nbuf
    npairs = ngroups // 2
    rps = npad // _NS
    zc = rps // _G

    def body(g_hbm, pk_hbm, out_hbm,
             acc_sh, bufs, pk_all, sidx, didx, gsem_a, gsem_b, ssem):
        c = lax.axis_index("c")
        s = lax.axis_index("s")
        tab = g_hbm

        pltpu.sync_copy(pk_hbm.at[c * _NS + s], pk_all)

        def zrow(i, carry):
            for j in range(d // 16):
                bufs[0, 0, i, pl.ds(j * 16, 16)] = jnp.zeros((16,), jnp.float32)
            return carry

        lax.fori_loop(0, _G, zrow, 0)
        for j in range(zc):
            pltpu.sync_copy(bufs.at[0, 0], acc_sh.at[pl.ds(s * rps + j * _G, _G)])
        plsc.subcore_barrier()

        gsems = (gsem_a, gsem_b)
        mask = jnp.int32(0xFFFF)

        def unpack(sl, grp):
            for b in range(nbuf):
                q = grp * nbuf + b
                for j in range(_G // 16):
                    v = pk_all[q, pl.ds(j * 16, 16)]
                    sidx[sl, b, pl.ds(j * 16, 16)] = v & mask
                    didx[sl, b, pl.ds(j * 16, 16)] = lax.shift_right_logical(v, 16)

        def issue_gathers(sl, grp):
            unpack(sl, grp)
            for b in range(nbuf):
                pltpu.async_copy(tab.at[sidx.at[sl, b]], bufs.at[sl, b], gsems[sl])

        def drain_gathers(sl):
            for b in range(nbuf):
                pltpu.make_async_copy(tab.at[sidx.at[sl, b]],
                                      bufs.at[sl, b], gsems[sl]).wait()

        def run_scatters(sl):
            descs = [pltpu.async_copy(bufs.at[sl, b], acc_sh.at[didx.at[sl, b]],
                                      ssem, add=True)
                     for b in range(nbuf)]
            for p_ in descs:
                p_.wait()

        issue_gathers(0, 0)

        def pair(h, carry):
            ga = 2 * h
            gb = 2 * h + 1
            issue_gathers(1, gb)
            drain_gathers(0)
            run_scatters(0)
            issue_gathers(0, (ga + 2) % ngroups)
            drain_gathers(1)
            run_scatters(1)
            return carry

        lax.fori_loop(0, npairs, pair, 0)
        drain_gathers(0)
        plsc.subcore_barrier()
        for j in range(zc):
            r0 = s * rps + j * _G
            pltpu.sync_copy(acc_sh.at[pl.ds(r0, _G)], bufs.at[0, 0])
            pltpu.sync_copy(bufs.at[0, 0], out_hbm.at[c].at[pl.ds(r0, _G)])

    return pl.kernel(
        body,
        out_type=jax.ShapeDtypeStruct((_NC, npad, d), jnp.float32),
        mesh=_mesh(),
        scratch_types=[
            pltpu.VMEM_SHARED((npad, d), jnp.float32),
            pltpu.VMEM((2, nbuf, _G, d), jnp.float32),
            pltpu.VMEM((chunks, _G), jnp.int32),
            pltpu.VMEM((2, nbuf, _G), jnp.int32),
            pltpu.VMEM((2, nbuf, _G), jnp.int32),
            pltpu.SemaphoreType.DMA,
            pltpu.SemaphoreType.DMA,
            pltpu.SemaphoreType.DMA,
        ],
        compiler_params=pltpu.CompilerParams(
            use_tc_tiling_on_sc=(d >= 128),
            disable_bounds_checks=True),
        name=f"gcn_agg_{npad}_{ep}_{d}",
    )


_BR = 2048


def _row_spec(br, w):
    return pl.BlockSpec((br, w), lambda i: (i, 0))


def _full_spec(shape):
    return pl.BlockSpec(shape, lambda i: (0,) * len(shape))


def _stage_a(d0, d1, xp):
    npad, f = xp.shape

    def body(d0_r, d1_r, x_r, dis_r, g1_r):
        dis = lax.rsqrt(d0_r[...] + d1_r[...] + 1.0)
        dis_r[...] = dis
        g1_r[...] = x_r[...] * dis

    return pl.pallas_call(
        body,
        grid=(npad // _BR,),
        in_specs=[_row_spec(_BR, 1), _row_spec(_BR, 1), _row_spec(_BR, f)],
        out_specs=[_row_spec(_BR, 1), _row_spec(_BR, f)],
        out_shape=[jax.ShapeDtypeStruct((npad, 1), jnp.float32),
                   jax.ShapeDtypeStruct((npad, f), jnp.float32)],
    )(d0, d1, xp)


def _pair_spec(w):
    return pl.BlockSpec((2, _BR, w), lambda i: (0, i, 0))


def _stage_b(p, g1, dis, W1, b1, W2):
    npad, f = g1.shape
    h1w = W1.shape[1]
    h2w = W2.shape[1]

    def body(p_r, g1_r, dis_r, w1_r, b1_r, w2_r, g2_r):
        ds_ = dis_r[...]
        pv = p_r[...]
        z1 = ds_ * (pv[0] + pv[1] + g1_r[...])
        h1 = jnp.dot(z1, w1_r[...], preferred_element_type=jnp.float32)
        h1 = jnp.maximum(h1 + b1_r[...], 0.0)
        t2 = jnp.dot(h1, w2_r[...], preferred_element_type=jnp.float32)
        g2_r[...] = ds_ * t2

    return pl.pallas_call(
        body,
        grid=(npad // _BR,),
        in_specs=[_pair_spec(f), _row_spec(_BR, f),
                  _row_spec(_BR, 1), _full_spec((f, h1w)),
                  _full_spec((1, h1w)), _full_spec((h1w, h2w))],
        out_specs=_row_spec(_BR, h2w),
        out_shape=jax.ShapeDtypeStruct((npad, h2w), jnp.float32),
    )(p, g1, dis, W1, b1, W2)


def _stage_c(p, g2, dis, b2, W3):
    npad, h2w = g2.shape
    cw = W3.shape[1]

    def body(p_r, g2_r, dis_r, b2_r, w3_r, g3_r):
        pv = p_r[...]
        h2 = dis_r[...] * (pv[0] + pv[1] + g2_r[...]) + b2_r[...]
        h2 = jnp.maximum(h2, 0.0)
        t3 = jnp.dot(h2, w3_r[...], preferred_element_type=jnp.float32)
        g3_r[...] = dis_r[...] * t3

    return pl.pallas_call(
        body,
        grid=(npad // _BR,),
        in_specs=[_pair_spec(h2w), _row_spec(_BR, h2w),
                  _row_spec(_BR, 1), _full_spec((1, h2w)), _full_spec((h2w, cw))],
        out_specs=_row_spec(_BR, cw),
        out_shape=jax.ShapeDtypeStruct((npad, cw), jnp.float32),
    )(p, g2, dis, b2, W3)


def _stage_d(p, g3, dis, b3):
    npad, cw = g3.shape

    def body(p_r, g3_r, dis_r, b3_r, out_r):
        pv = p_r[...]
        out_r[...] = dis_r[...] * (pv[0] + pv[1] + g3_r[...]) + b3_r[...]

    return pl.pallas_call(
        body,
        grid=(npad // _BR,),
        in_specs=[_pair_spec(cw), _row_spec(_BR, cw),
                  _row_spec(_BR, 1), _full_spec((1, cw))],
        out_specs=_row_spec(_BR, cw),
        out_shape=jax.ShapeDtypeStruct((npad, cw), jnp.float32),
    )(p, g3, dis, b3)


def kernel(x, edge_index, W1, b1, W2, b2, W3, b3):
    n, f = x.shape
    e = edge_index.shape[1]

    npad = ((n + 16 + _BR - 1) // _BR) * _BR
    step = _NW * _G
    chunks = (e + step - 1) // step
    chunks = ((chunks + 15) // 16) * 16
    ep = chunks * step

    spare = npad - n
    pad_idx = n + (jnp.arange(ep - e, dtype=jnp.int32) % spare)
    src_flat = jnp.concatenate([edge_index[0], pad_idx])
    dst_flat = jnp.concatenate([edge_index[1], pad_idx])
    pkp = (src_flat | (dst_flat << 16)).reshape(_NW, chunks, _G)
    xp = jnp.pad(x, ((0, npad - n), (0, 0)))
    fh = f // 2

    degs = _make_deg(npad, ep)(pkp)
    d0 = degs[:npad].reshape(npad, 1)
    d1 = degs[npad:].reshape(npad, 1)
    dis, g1 = _stage_a(d0, d1, xp)
    p = _make_agg(npad, ep, f)(g1, pkp)
    g2 = _stage_b(p, g1, dis, W1, b1.reshape(1, -1), W2)

    p = _make_agg(npad, ep, g2.shape[1])(g2, pkp)
    g3 = _stage_c(p, g2, dis, b2.reshape(1, -1), W3)

    p = _make_agg(npad, ep, g3.shape[1])(g3, pkp)
    out = _stage_d(p, g3, dis, b3.reshape(1, -1))
    return out[:n]


# speedup vs baseline: 1.0620x; 1.0018x over previous
import functools

import jax
import jax.numpy as jnp
from jax import lax
from jax.experimental import pallas as pl
from jax.experimental.pallas import tpu as pltpu
import jax.experimental.pallas.tpu_sc as plsc

_NC = 2
_NS = 16
_NW = _NC * _NS
_G = 128


def _mesh():
    return plsc.VectorSubcoreMesh(
        core_axis_name="c", subcore_axis_name="s",
        num_cores=_NC, num_subcores=_NS)


@functools.cache
def _make_deg(npad, ep):
    chunks = ep // (_NW * _G)
    rps = npad // _NS
    zc = rps // _G

    def body(pk_hbm, out_hbm, acc_sh, ones_v, buf_v, dst_all, ssem):
        c = lax.axis_index("c")
        s = lax.axis_index("s")
        wid = c * _NS + s
        pltpu.sync_copy(pk_hbm.at[wid], dst_all)

        def unpk(k, carry):
            for j in range(_G // 16):
                v = dst_all[k, pl.ds(j * 16, 16)]
                dst_all[k, pl.ds(j * 16, 16)] = lax.shift_right_logical(v, 16)
            return carry

        lax.fori_loop(0, chunks, unpk, 0)
        for i in range(_G // 16):
            ones_v[pl.ds(i * 16, 16)] = jnp.ones((16,), jnp.float32)
            buf_v[pl.ds(i * 16, 16)] = jnp.zeros((16,), jnp.float32)
        for j in range(zc):
            pltpu.sync_copy(buf_v, acc_sh.at[pl.ds(s * rps + j * _G, _G)])
        plsc.subcore_barrier()

        def issue(k, carry):
            pltpu.async_copy(ones_v, acc_sh.at[dst_all.at[k]], ssem, add=True)
            return carry

        lax.fori_loop(0, chunks, issue, 0)

        def drain(k, carry):
            pltpu.make_async_copy(ones_v, acc_sh.at[dst_all.at[k]], ssem).wait()
            return carry

        lax.fori_loop(0, chunks, drain, 0)
        plsc.subcore_barrier()
        for j in range(zc):
            r0 = s * rps + j * _G
            pltpu.sync_copy(acc_sh.at[pl.ds(r0, _G)], buf_v)
            pltpu.sync_copy(buf_v, out_hbm.at[pl.ds(c * npad + r0, _G)])

    return pl.kernel(
        body,
        out_type=jax.ShapeDtypeStruct((_NC * npad,), jnp.float32),
        mesh=_mesh(),
        scratch_types=[
            pltpu.VMEM_SHARED((npad,), jnp.float32),
            pltpu.VMEM((_G,), jnp.float32),
            pltpu.VMEM((_G,), jnp.float32),
            pltpu.VMEM((ep // (_NW * _G), _G), jnp.int32),
            pltpu.SemaphoreType.DMA,
        ],
        name=f"gcn_deg_{npad}_{ep}",
    )


@functools.cache
def _make_agg(npad, ep, d):
    chunks = ep // (_NW * _G)
    nbuf = 1 if d >= 128 else 8
    ngroups = chunks // nbuf
    npairs = ngroups // 2
    rps = npad // _NS
    zc = rps // _G

    def body(g_hbm, pk_hbm, out_hbm,
             acc_sh, bufs, pk_all, sidx, didx, gsem_a, gsem_b, ssem):
        c = lax.axis_index("c")
        s = lax.axis_index("s")
        tab = g_hbm

        pltpu.sync_copy(pk_hbm.at[c * _NS + s], pk_all)

        def zrow(i, carry):
            for j in range(d // 16):
                bufs[0, 0, i, pl.ds(j * 16, 16)] = jnp.zeros((16,), jnp.float32)
            return carry

        lax.fori_loop(0, _G, zrow, 0)
        for j in range(zc):
            pltpu.sync_copy(bufs.at[0, 0], acc_sh.at[pl.ds(s * rps + j * _G, _G)])
        plsc.subcore_barrier()

        gsems = (gsem_a, gsem_b)
        mask = jnp.int32(0xFFFF)

        def unpack(sl, grp):
            for b in range(nbuf):
                q = grp * nbuf + b
                for j in range(_G // 16):
                    v = pk_all[q, pl.ds(j * 16, 16)]
                    sidx[sl, b, pl.ds(j * 16, 16)] = v & mask
                    didx[sl, b, pl.ds(j * 16, 16)] = lax.shift_right_logical(v, 16)

        def issue_gathers(sl, grp):
            unpack(sl, grp)
            for b in range(nbuf):
                pltpu.async_copy(tab.at[sidx.at[sl, b]], bufs.at[sl, b], gsems[sl])

        def drain_gathers(sl):
            for b in range(nbuf):
                pltpu.make_async_copy(tab.at[sidx.at[sl, b]],
                                      bufs.at[sl, b], gsems[sl]).wait()

        def run_scatters(sl):
            descs = [pltpu.async_copy(bufs.at[sl, b], acc_sh.at[didx.at[sl, b]],
                                      ssem, add=True)
                     for b in range(nbuf)]
            for p_ in descs:
                p_.wait()

        issue_gathers(0, 0)

        def pair(h, carry):
            ga = 2 * h
            gb = 2 * h + 1
            issue_gathers(1, gb)
            drain_gathers(0)
            run_scatters(0)
            issue_gathers(0, (ga + 2) % ngroups)
            drain_gathers(1)
            run_scatters(1)
            return carry

        lax.fori_loop(0, npairs, pair, 0)
        drain_gathers(0)
        plsc.subcore_barrier()
        for j in range(zc):
            r0 = s * rps + j * _G
            pltpu.sync_copy(acc_sh.at[pl.ds(r0, _G)], bufs.at[0, 0])
            pltpu.sync_copy(bufs.at[0, 0], out_hbm.at[c].at[pl.ds(r0, _G)])

    return pl.kernel(
        body,
        out_type=jax.ShapeDtypeStruct((_NC, npad, d), jnp.float32),
        mesh=_mesh(),
        scratch_types=[
            pltpu.VMEM_SHARED((npad, d), jnp.float32),
            pltpu.VMEM((2, nbuf, _G, d), jnp.float32),
            pltpu.VMEM((chunks, _G), jnp.int32),
            pltpu.VMEM((2, nbuf, _G), jnp.int32),
            pltpu.VMEM((2, nbuf, _G), jnp.int32),
            pltpu.SemaphoreType.DMA,
            pltpu.SemaphoreType.DMA,
            pltpu.SemaphoreType.DMA,
        ],
        compiler_params=pltpu.CompilerParams(
            use_tc_tiling_on_sc=(d >= 128),
            disable_bounds_checks=True),
        name=f"gcn_agg_{npad}_{ep}_{d}",
    )


_BR = 2048


def _row_spec(br, w):
    return pl.BlockSpec((br, w), lambda i: (i, 0))


def _full_spec(shape):
    return pl.BlockSpec(shape, lambda i: (0,) * len(shape))


def _stage_a(d0, d1, x, npad):
    f = x.shape[1]

    def body(d0_r, d1_r, x_r, dis_r, g1_r):
        dis = lax.rsqrt(d0_r[...] + d1_r[...] + 1.0)
        dis_r[...] = dis
        g1_r[...] = x_r[...] * dis

    return pl.pallas_call(
        body,
        grid=(npad // _BR,),
        in_specs=[_row_spec(_BR, 1), _row_spec(_BR, 1), _row_spec(_BR, f)],
        out_specs=[_row_spec(_BR, 1), _row_spec(_BR, f)],
        out_shape=[jax.ShapeDtypeStruct((npad, 1), jnp.float32),
                   jax.ShapeDtypeStruct((npad, f), jnp.float32)],
    )(d0, d1, x)


def _pair_spec(w):
    return pl.BlockSpec((2, _BR, w), lambda i: (0, i, 0))


def _stage_b(p, g1, dis, W1, b1, W2):
    npad, f = g1.shape
    h1w = W1.shape[1]
    h2w = W2.shape[1]

    def body(p_r, g1_r, dis_r, w1_r, b1_r, w2_r, g2_r):
        ds_ = dis_r[...]
        pv = p_r[...]
        z1 = ds_ * (pv[0] + pv[1] + g1_r[...])
        h1 = jnp.dot(z1, w1_r[...], preferred_element_type=jnp.float32)
        h1 = jnp.maximum(h1 + b1_r[...], 0.0)
        t2 = jnp.dot(h1, w2_r[...], preferred_element_type=jnp.float32)
        g2_r[...] = ds_ * t2

    return pl.pallas_call(
        body,
        grid=(npad // _BR,),
        in_specs=[_pair_spec(f), _row_spec(_BR, f),
                  _row_spec(_BR, 1), _full_spec((f, h1w)),
                  _full_spec((1, h1w)), _full_spec((h1w, h2w))],
        out_specs=_row_spec(_BR, h2w),
        out_shape=jax.ShapeDtypeStruct((npad, h2w), jnp.float32),
    )(p, g1, dis, W1, b1, W2)


def _stage_c(p, g2, dis, b2, W3):
    npad, h2w = g2.shape
    cw = W3.shape[1]

    def body(p_r, g2_r, dis_r, b2_r, w3_r, g3_r):
        pv = p_r[...]
        h2 = dis_r[...] * (pv[0] + pv[1] + g2_r[...]) + b2_r[...]
        h2 = jnp.maximum(h2, 0.0)
        t3 = jnp.dot(h2, w3_r[...], preferred_element_type=jnp.float32)
        g3_r[...] = dis_r[...] * t3

    return pl.pallas_call(
        body,
        grid=(npad // _BR,),
        in_specs=[_pair_spec(h2w), _row_spec(_BR, h2w),
                  _row_spec(_BR, 1), _full_spec((1, h2w)), _full_spec((h2w, cw))],
        out_specs=_row_spec(_BR, cw),
        out_shape=jax.ShapeDtypeStruct((npad, cw), jnp.float32),
    )(p, g2, dis, b2, W3)


def _stage_d(p, g3, dis, b3):
    npad, cw = g3.shape

    def body(p_r, g3_r, dis_r, b3_r, out_r):
        pv = p_r[...]
        out_r[...] = dis_r[...] * (pv[0] + pv[1] + g3_r[...]) + b3_r[...]

    return pl.pallas_call(
        body,
        grid=(npad // _BR,),
        in_specs=[_pair_spec(cw), _row_spec(_BR, cw),
                  _row_spec(_BR, 1), _full_spec((1, cw))],
        out_specs=_row_spec(_BR, cw),
        out_shape=jax.ShapeDtypeStruct((npad, cw), jnp.float32),
    )(p, g3, dis, b3)


def kernel(x, edge_index, W1, b1, W2, b2, W3, b3):
    n, f = x.shape
    e = edge_index.shape[1]

    npad = ((n + 16 + _BR - 1) // _BR) * _BR
    step = _NW * _G
    chunks = (e + step - 1) // step
    chunks = ((chunks + 15) // 16) * 16
    ep = chunks * step

    spare = npad - n
    pad_idx = n + (jnp.arange(ep - e, dtype=jnp.int32) % spare)
    src_flat = jnp.concatenate([edge_index[0], pad_idx])
    dst_flat = jnp.concatenate([edge_index[1], pad_idx])
    pkp = (src_flat | (dst_flat << 16)).reshape(_NW, chunks, _G)
    fh = f // 2

    degs = _make_deg(npad, ep)(pkp)
    d0 = degs[:npad].reshape(npad, 1)
    d1 = degs[npad:].reshape(npad, 1)
    dis, g1 = _stage_a(d0, d1, x, npad)
    p = _make_agg(npad, ep, f)(g1, pkp)
    g2 = _stage_b(p, g1, dis, W1, b1.reshape(1, -1), W2)

    p = _make_agg(npad, ep, g2.shape[1])(g2, pkp)
    g3 = _stage_c(p, g2, dis, b2.reshape(1, -1), W3)

    p = _make_agg(npad, ep, g3.shape[1])(g3, pkp)
    out = _stage_d(p, g3, dis, b3.reshape(1, -1))
    return out[:n]
